# Initial kernel scaffold; baseline (speedup 1.0000x reference)
import jax
import jax.numpy as jnp
from jax.experimental import pallas as pl


def kernel(inputs, rnn_hxs, masks, controller_ids, AW1, Ab1, AW2, Ab2, CW1, Cb1, CW2, Cb2, VW, Vb):
    raise NotImplementedError("write your pallas kernel here")


# speedup vs baseline: 1.4209x; 1.4209x over previous
import functools

import jax
import jax.numpy as jnp
from jax.experimental import pallas as pl
from jax.experimental.pallas import tpu as pltpu


BT = 128


def _tc_body(be_ref, x_ref, w1a, b1a, w2a, b2a, w1c, b1c, w2c, b2c, vw, vb,
             y_ref, v_ref):
    x = x_ref[...]
    f32 = jnp.float32
    h = jnp.tanh(jnp.dot(x, w1a[0], preferred_element_type=f32) + b1a[0])
    ha = jnp.tanh(jnp.dot(h, w2a[0], preferred_element_type=f32) + b2a[0])
    y_ref[...] = ha
    g = jnp.tanh(jnp.dot(x, w1c[0], preferred_element_type=f32) + b1c[0])
    hc = jnp.tanh(jnp.dot(g, w2c[0], preferred_element_type=f32) + b2c[0])
    v_ref[...] = jnp.dot(hc, vw[0], preferred_element_type=f32) + vb[0]


def _bias_spec(H):
    return pl.BlockSpec((1, 1, H), lambda i, be: (be[i], 0, 0))


def _expert_mlp(x_padded, block_expert, AW1, Ab1, AW2, Ab2, CW1, Cb1, CW2, Cb2,
                VWp, Vbp):
    P, D = x_padded.shape
    E, _, H = AW1.shape
    nbp = P // BT
    grid_spec = pltpu.PrefetchScalarGridSpec(
        num_scalar_prefetch=1,
        grid=(nbp,),
        in_specs=[
            pl.BlockSpec((BT, D), lambda i, be: (i, 0)),
            pl.BlockSpec((1, D, H), lambda i, be: (be[i], 0, 0)),
            _bias_spec(H),
            pl.BlockSpec((1, H, H), lambda i, be: (be[i], 0, 0)),
            _bias_spec(H),
            pl.BlockSpec((1, D, H), lambda i, be: (be[i], 0, 0)),
            _bias_spec(H),
            pl.BlockSpec((1, H, H), lambda i, be: (be[i], 0, 0)),
            _bias_spec(H),
            pl.BlockSpec((1, H, 128), lambda i, be: (be[i], 0, 0)),
            _bias_spec(128),
        ],
        out_specs=[
            pl.BlockSpec((BT, H), lambda i, be: (i, 0)),
            pl.BlockSpec((BT, 128), lambda i, be: (i, 0)),
        ],
    )
    return pl.pallas_call(
        _tc_body,
        grid_spec=grid_spec,
        out_shape=[
            jax.ShapeDtypeStruct((P, H), jnp.float32),
            jax.ShapeDtypeStruct((P, 128), jnp.float32),
        ],
    )(block_expert, x_padded, AW1, Ab1, AW2, Ab2, CW1, Cb1, CW2, Cb2, VWp, Vbp)


def kernel(inputs, rnn_hxs, masks, controller_ids, AW1, Ab1, AW2, Ab2,
           CW1, Cb1, CW2, Cb2, VW, Vb):
    B, D = inputs.shape
    E, _, H = AW1.shape
    P = B + E * BT
    nbp = P // BT

    ids = controller_ids.astype(jnp.int32)
    onehot = (ids[:, None] == jnp.arange(E, dtype=jnp.int32)[None, :])
    occ = jnp.cumsum(onehot.astype(jnp.int32), axis=0)
    counts = occ[-1]
    padded = ((counts + BT - 1) // BT) * BT
    pad_starts = jnp.cumsum(padded) - padded
    rank_in_e = jnp.take_along_axis(occ, ids[:, None], axis=1)[:, 0] - 1
    pos = pad_starts[ids] + rank_in_e
    gidx = jnp.zeros((P,), jnp.int32).at[pos].set(
        jnp.arange(B, dtype=jnp.int32))
    pad_ends = jnp.cumsum(padded)
    block_expert = jnp.minimum(
        jnp.searchsorted(pad_ends, jnp.arange(nbp, dtype=jnp.int32) * BT,
                         side="right"),
        E - 1).astype(jnp.int32)

    x_padded = jnp.take(inputs, gidx, axis=0)

    VWp = jnp.pad(VW, ((0, 0), (0, 0), (0, 128 - VW.shape[-1])))
    Vbp = jnp.pad(Vb, ((0, 0), (0, 128 - Vb.shape[-1]))).reshape(E, 1, 128)
    y_padded, v_padded = _expert_mlp(
        x_padded, block_expert,
        AW1, Ab1.reshape(E, 1, H), AW2, Ab2.reshape(E, 1, H),
        CW1, Cb1.reshape(E, 1, H), CW2, Cb2.reshape(E, 1, H),
        VWp, Vbp)

    actor_features = jnp.take(y_padded, pos, axis=0)
    value = jnp.take(v_padded, pos, axis=0)[:, :1]
    return value, actor_features, rnn_hxs


# speedup vs baseline: 1.5265x; 1.0743x over previous
import functools

import jax
import jax.numpy as jnp
from jax import lax
from jax.experimental import pallas as pl
from jax.experimental.pallas import tpu as pltpu
from jax.experimental.pallas import tpu_sc as plsc


BT = 128

_SC_INFO = plsc.get_sparse_core_info()
_NW = _SC_INFO.num_cores * _SC_INFO.num_subcores


def _sc_gather(table, idx):
    n = idx.shape[0]
    d = table.shape[1]
    bpw = n // _NW
    mesh = plsc.VectorSubcoreMesh(core_axis_name="c", subcore_axis_name="s")

    @functools.partial(
        pl.kernel, mesh=mesh,
        out_type=jax.ShapeDtypeStruct((n, d), table.dtype),
        scratch_types=[
            pltpu.VMEM((bpw,), jnp.int32),
            pltpu.VMEM((bpw, d), table.dtype),
            pltpu.SemaphoreType.DMA,
        ],
    )
    def k(table_hbm, idx_hbm, out_hbm, idx_v, rows_v, sem):
        wid = lax.axis_index("s") * _SC_INFO.num_cores + lax.axis_index("c")
        base = wid * bpw
        pltpu.sync_copy(idx_hbm.at[pl.ds(base, bpw)], idx_v)
        pltpu.async_copy(table_hbm.at[idx_v], rows_v, sem).wait()
        pltpu.sync_copy(rows_v, out_hbm.at[pl.ds(base, bpw)])

    return k(table, idx)


def _sc_gather2(tab_a, tab_b, idx):
    n = idx.shape[0]
    da, db = tab_a.shape[1], tab_b.shape[1]
    bpw = n // _NW
    mesh = plsc.VectorSubcoreMesh(core_axis_name="c", subcore_axis_name="s")

    @functools.partial(
        pl.kernel, mesh=mesh,
        out_type=[
            jax.ShapeDtypeStruct((n, da), tab_a.dtype),
            jax.ShapeDtypeStruct((n, db), tab_b.dtype),
        ],
        scratch_types=[
            pltpu.VMEM((bpw,), jnp.int32),
            pltpu.VMEM((bpw, da), tab_a.dtype),
            pltpu.VMEM((bpw, db), tab_b.dtype),
            pltpu.SemaphoreType.DMA,
        ],
    )
    def k(a_hbm, b_hbm, idx_hbm, out_a, out_b, idx_v, rows_a, rows_b, sem):
        wid = lax.axis_index("s") * _SC_INFO.num_cores + lax.axis_index("c")
        base = wid * bpw
        pltpu.sync_copy(idx_hbm.at[pl.ds(base, bpw)], idx_v)
        cp_a = pltpu.async_copy(a_hbm.at[idx_v], rows_a, sem)
        cp_b = pltpu.async_copy(b_hbm.at[idx_v], rows_b, sem)
        cp_a.wait()
        cp_b.wait()
        pltpu.sync_copy(rows_a, out_a.at[pl.ds(base, bpw)])
        pltpu.sync_copy(rows_b, out_b.at[pl.ds(base, bpw)])

    return k(tab_a, tab_b, idx)


def _tc_body(be_ref, x_ref, w1a, b1a, w2a, b2a, w1c, b1c, w2c, b2c, vw, vb,
             y_ref, v_ref):
    x = x_ref[...]
    f32 = jnp.float32
    h = jnp.tanh(jnp.dot(x, w1a[0], preferred_element_type=f32) + b1a[0])
    ha = jnp.tanh(jnp.dot(h, w2a[0], preferred_element_type=f32) + b2a[0])
    y_ref[...] = ha
    g = jnp.tanh(jnp.dot(x, w1c[0], preferred_element_type=f32) + b1c[0])
    hc = jnp.tanh(jnp.dot(g, w2c[0], preferred_element_type=f32) + b2c[0])
    v_ref[...] = jnp.dot(hc, vw[0], preferred_element_type=f32) + vb[0]


def _bias_spec(H):
    return pl.BlockSpec((1, 1, H), lambda i, be: (be[i], 0, 0))


def _expert_mlp(x_padded, block_expert, AW1, Ab1, AW2, Ab2, CW1, Cb1, CW2, Cb2,
                VWp, Vbp):
    P, D = x_padded.shape
    E, _, H = AW1.shape
    nbp = P // BT
    grid_spec = pltpu.PrefetchScalarGridSpec(
        num_scalar_prefetch=1,
        grid=(nbp,),
        in_specs=[
            pl.BlockSpec((BT, D), lambda i, be: (i, 0)),
            pl.BlockSpec((1, D, H), lambda i, be: (be[i], 0, 0)),
            _bias_spec(H),
            pl.BlockSpec((1, H, H), lambda i, be: (be[i], 0, 0)),
            _bias_spec(H),
            pl.BlockSpec((1, D, H), lambda i, be: (be[i], 0, 0)),
            _bias_spec(H),
            pl.BlockSpec((1, H, H), lambda i, be: (be[i], 0, 0)),
            _bias_spec(H),
            pl.BlockSpec((1, H, 128), lambda i, be: (be[i], 0, 0)),
            _bias_spec(128),
        ],
        out_specs=[
            pl.BlockSpec((BT, H), lambda i, be: (i, 0)),
            pl.BlockSpec((BT, 128), lambda i, be: (i, 0)),
        ],
    )
    return pl.pallas_call(
        _tc_body,
        grid_spec=grid_spec,
        out_shape=[
            jax.ShapeDtypeStruct((P, H), jnp.float32),
            jax.ShapeDtypeStruct((P, 128), jnp.float32),
        ],
    )(block_expert, x_padded, AW1, Ab1, AW2, Ab2, CW1, Cb1, CW2, Cb2, VWp, Vbp)


def kernel(inputs, rnn_hxs, masks, controller_ids, AW1, Ab1, AW2, Ab2,
           CW1, Cb1, CW2, Cb2, VW, Vb):
    B, D = inputs.shape
    E, _, H = AW1.shape
    P = B + E * BT
    nbp = P // BT

    ids = controller_ids.astype(jnp.int32)
    onehot = (ids[:, None] == jnp.arange(E, dtype=jnp.int32)[None, :])
    occ = jnp.cumsum(onehot.astype(jnp.int32), axis=0)
    counts = occ[-1]
    padded = ((counts + BT - 1) // BT) * BT
    pad_starts = jnp.cumsum(padded) - padded
    rank_in_e = jnp.take_along_axis(occ, ids[:, None], axis=1)[:, 0] - 1
    pos = pad_starts[ids] + rank_in_e
    gidx = jnp.zeros((P,), jnp.int32).at[pos].set(
        jnp.arange(B, dtype=jnp.int32))
    pad_ends = jnp.cumsum(padded)
    block_expert = jnp.minimum(
        jnp.searchsorted(pad_ends, jnp.arange(nbp, dtype=jnp.int32) * BT,
                         side="right"),
        E - 1).astype(jnp.int32)

    x_padded = _sc_gather(inputs, gidx)

    VWp = jnp.pad(VW, ((0, 0), (0, 0), (0, 128 - VW.shape[-1])))
    Vbp = jnp.pad(Vb, ((0, 0), (0, 128 - Vb.shape[-1]))).reshape(E, 1, 128)
    y_padded, v_padded = _expert_mlp(
        x_padded, block_expert,
        AW1, Ab1.reshape(E, 1, H), AW2, Ab2.reshape(E, 1, H),
        CW1, Cb1.reshape(E, 1, H), CW2, Cb2.reshape(E, 1, H),
        VWp, Vbp)

    actor_features, value = _sc_gather2(y_padded, v_padded, pos)
    return value[:, :1], actor_features, rnn_hxs


# speedup vs baseline: 2.5857x; 1.6939x over previous
import functools

import jax
import jax.numpy as jnp
from jax import lax
from jax.experimental import pallas as pl
from jax.experimental.pallas import tpu as pltpu
from jax.experimental.pallas import tpu_sc as plsc


BT = 128

_SC_INFO = plsc.get_sparse_core_info()
_NW = _SC_INFO.num_cores * _SC_INFO.num_subcores


def _sc_scatter(src, pos, P):
    n, d = src.shape
    bpw = n // _NW
    mesh = plsc.VectorSubcoreMesh(core_axis_name="c", subcore_axis_name="s")

    @functools.partial(
        pl.kernel, mesh=mesh,
        out_type=jax.ShapeDtypeStruct((P, d), src.dtype),
        scratch_types=[
            pltpu.VMEM((bpw,), jnp.int32),
            pltpu.VMEM((bpw, d), src.dtype),
            pltpu.SemaphoreType.DMA,
        ],
    )
    def k(src_hbm, pos_hbm, out_hbm, idx_v, rows_v, sem):
        wid = lax.axis_index("s") * _SC_INFO.num_cores + lax.axis_index("c")
        base = wid * bpw
        pltpu.sync_copy(pos_hbm.at[pl.ds(base, bpw)], idx_v)
        pltpu.sync_copy(src_hbm.at[pl.ds(base, bpw)], rows_v)
        pltpu.async_copy(rows_v, out_hbm.at[idx_v], sem).wait()

    return k(src, pos)


def _sc_gather2(tab_a, tab_b, idx):
    n = idx.shape[0]
    da, db = tab_a.shape[1], tab_b.shape[1]
    bpw = n // _NW
    mesh = plsc.VectorSubcoreMesh(core_axis_name="c", subcore_axis_name="s")

    @functools.partial(
        pl.kernel, mesh=mesh,
        out_type=[
            jax.ShapeDtypeStruct((n, da), tab_a.dtype),
            jax.ShapeDtypeStruct((n, db), tab_b.dtype),
        ],
        scratch_types=[
            pltpu.VMEM((bpw,), jnp.int32),
            pltpu.VMEM((bpw, da), tab_a.dtype),
            pltpu.VMEM((bpw, db), tab_b.dtype),
            pltpu.SemaphoreType.DMA,
        ],
    )
    def k(a_hbm, b_hbm, idx_hbm, out_a, out_b, idx_v, rows_a, rows_b, sem):
        wid = lax.axis_index("s") * _SC_INFO.num_cores + lax.axis_index("c")
        base = wid * bpw
        pltpu.sync_copy(idx_hbm.at[pl.ds(base, bpw)], idx_v)
        cp_a = pltpu.async_copy(a_hbm.at[idx_v], rows_a, sem)
        cp_b = pltpu.async_copy(b_hbm.at[idx_v], rows_b, sem)
        cp_a.wait()
        cp_b.wait()
        pltpu.sync_copy(rows_a, out_a.at[pl.ds(base, bpw)])
        pltpu.sync_copy(rows_b, out_b.at[pl.ds(base, bpw)])

    return k(tab_a, tab_b, idx)


def _tc_body(be_ref, x_ref, w1a, b1a, w2a, b2a, w1c, b1c, w2c, b2c, vw, vb,
             y_ref, v_ref):
    x = x_ref[...]
    f32 = jnp.float32
    h = jnp.tanh(jnp.dot(x, w1a[0], preferred_element_type=f32) + b1a[0])
    ha = jnp.tanh(jnp.dot(h, w2a[0], preferred_element_type=f32) + b2a[0])
    y_ref[...] = ha
    g = jnp.tanh(jnp.dot(x, w1c[0], preferred_element_type=f32) + b1c[0])
    hc = jnp.tanh(jnp.dot(g, w2c[0], preferred_element_type=f32) + b2c[0])
    v_ref[...] = jnp.dot(hc, vw[0], preferred_element_type=f32) + vb[0]


def _bias_spec(H):
    return pl.BlockSpec((1, 1, H), lambda i, be: (be[i], 0, 0))


def _expert_mlp(x_padded, block_expert, AW1, Ab1, AW2, Ab2, CW1, Cb1, CW2, Cb2,
                VWp, Vbp):
    P, D = x_padded.shape
    E, _, H = AW1.shape
    nbp = P // BT
    grid_spec = pltpu.PrefetchScalarGridSpec(
        num_scalar_prefetch=1,
        grid=(nbp,),
        in_specs=[
            pl.BlockSpec((BT, D), lambda i, be: (i, 0)),
            pl.BlockSpec((1, D, H), lambda i, be: (be[i], 0, 0)),
            _bias_spec(H),
            pl.BlockSpec((1, H, H), lambda i, be: (be[i], 0, 0)),
            _bias_spec(H),
            pl.BlockSpec((1, D, H), lambda i, be: (be[i], 0, 0)),
            _bias_spec(H),
            pl.BlockSpec((1, H, H), lambda i, be: (be[i], 0, 0)),
            _bias_spec(H),
            pl.BlockSpec((1, H, 128), lambda i, be: (be[i], 0, 0)),
            _bias_spec(128),
        ],
        out_specs=[
            pl.BlockSpec((BT, H), lambda i, be: (i, 0)),
            pl.BlockSpec((BT, 128), lambda i, be: (i, 0)),
        ],
    )
    return pl.pallas_call(
        _tc_body,
        grid_spec=grid_spec,
        out_shape=[
            jax.ShapeDtypeStruct((P, H), jnp.float32),
            jax.ShapeDtypeStruct((P, 128), jnp.float32),
        ],
    )(block_expert, x_padded, AW1, Ab1, AW2, Ab2, CW1, Cb1, CW2, Cb2, VWp, Vbp)


def kernel(inputs, rnn_hxs, masks, controller_ids, AW1, Ab1, AW2, Ab2,
           CW1, Cb1, CW2, Cb2, VW, Vb):
    B, D = inputs.shape
    E, _, H = AW1.shape
    P = B + E * BT
    nbp = P // BT

    ids = controller_ids.astype(jnp.int32)
    onehot = (ids[:, None] == jnp.arange(E, dtype=jnp.int32)[None, :]
              ).astype(jnp.int32)
    occ = jnp.cumsum(onehot, axis=0)
    counts = occ[-1]
    padded = ((counts + BT - 1) // BT) * BT
    pad_ends = jnp.cumsum(padded)
    pad_starts = pad_ends - padded
    rank_in_e = jnp.sum(occ * onehot, axis=1) - 1
    pos = jnp.sum(pad_starts[None, :] * onehot, axis=1) + rank_in_e
    block_expert = jnp.minimum(
        jnp.sum((jnp.arange(nbp, dtype=jnp.int32)[:, None] * BT
                 >= pad_ends[None, :]).astype(jnp.int32), axis=1),
        E - 1)

    x_padded = _sc_scatter(inputs, pos, P)

    VWp = jnp.pad(VW, ((0, 0), (0, 0), (0, 128 - VW.shape[-1])))
    Vbp = jnp.pad(Vb, ((0, 0), (0, 128 - Vb.shape[-1]))).reshape(E, 1, 128)
    y_padded, v_padded = _expert_mlp(
        x_padded, block_expert,
        AW1, Ab1.reshape(E, 1, H), AW2, Ab2.reshape(E, 1, H),
        CW1, Cb1.reshape(E, 1, H), CW2, Cb2.reshape(E, 1, H),
        VWp, Vbp)

    actor_features, value = _sc_gather2(y_padded, v_padded, pos)
    return value[:, :1], actor_features, rnn_hxs
